# baseline (speedup 1.0000x reference)
import functools

import numpy as np

import jax
import jax.numpy as jnp
from jax import lax
from jax.experimental import pallas as pl
from jax.experimental.pallas import tpu as pltpu
from jax.experimental.pallas import tpu_sc as plsc

D = 200
DLO = 128
DHI = D - DLO
S = 25
K = 3
RW = 80
RW2 = 2 * RW
B = 4096
NC, NS, L = 2, 16, 16
NW = NC * NS
PER_W = B // NW
NCH_LO = DLO // (2 * L)
NCH_HI = -(-DHI // (2 * L))


def _sc_scores(tlo, thi, wpad, lists):
    mesh = plsc.VectorSubcoreMesh(core_axis_name="c", subcore_axis_name="s")

    @functools.partial(
        pl.kernel,
        out_type=jax.ShapeDtypeStruct((NW, PER_W), jnp.float32),
        mesh=mesh,
        compiler_params=pltpu.CompilerParams(use_tc_tiling_on_sc=False,
                                             needs_layout_passes=False),
        scratch_types=[
            pltpu.VMEM((L,), jnp.float32),
            pltpu.VMEM((PER_W, RW2), jnp.int32),
            pltpu.VMEM((2 * RW2, DLO), jnp.bfloat16),
            pltpu.VMEM((2 * RW2, DLO), jnp.bfloat16),
            pltpu.VMEM((PER_W,), jnp.float32),
            pltpu.SemaphoreType.DMA,
            pltpu.SemaphoreType.DMA,
        ],
    )
    def body(tlo_hbm, thi_hbm, w_hbm, lists_hbm, out_hbm, w_v, idx_v, rows_a,
             rows_b, out_v, sem_a, sem_b):
        wid = lax.axis_index("s") * NC + lax.axis_index("c")
        pltpu.sync_copy(w_hbm, w_v)
        pltpu.sync_copy(lists_hbm.at[wid], idx_v)

        lanes = lax.iota(jnp.int32, L)
        zero = jnp.zeros((L,), jnp.float32)

        wv = w_v[...]
        e = jnp.exp(wv - jnp.max(wv))
        wn = e / jnp.full((L,), jnp.sum(e), jnp.float32)
        w0 = jnp.sum(jnp.where(lanes == 0, wn, zero))
        w1 = jnp.sum(jnp.where(lanes == 1, wn, zero)) * (1.0 / S)
        w2 = jnp.sum(jnp.where(lanes == 2, wn, zero)) * (1.0 / S)
        w3 = jnp.sum(jnp.where(lanes == 3, wn, zero)) * (1.0 / S)

        def issue(n, rows_v, sem):
            pltpu.async_copy(tlo_hbm.at[idx_v.at[n]],
                             rows_v.at[pl.ds(0, RW2)], sem)
            pltpu.async_copy(thi_hbm.at[idx_v.at[n]],
                             rows_v.at[pl.ds(RW2, RW2)], sem)

        def drain(rows_v, sem):
            pltpu.make_async_copy(tlo_hbm.at[pl.ds(0, 2 * RW2)], rows_v,
                                  sem).wait()

        nch = (NCH_LO, NCH_HI)

        def row_chunks(rows_v, lo_base, hi_base, j):
            out = []
            for t, base in ((0, lo_base), (1, hi_base)):
                for c in range(nch[t]):
                    bv = rows_v[base + j, pl.ds(c * 2 * L, 2 * L)]
                    out.append(plsc.bitcast(bv, jnp.int32))
            return out

        def unpack_acc(accs, chunks):
            res = list(accs)
            for i, v in enumerate(chunks):
                eo = (lax.bitcast_convert_type(lax.shift_left(v, 16),
                                               jnp.float32),
                      lax.bitcast_convert_type(
                          jnp.bitwise_and(v, jnp.int32(-65536)), jnp.float32))
                for h in range(2):
                    res[2 * i + h] = res[2 * i + h] + eo[h]
            return res

        NACC = 2 * (NCH_LO + NCH_HI)

        def side_repr(rows_v, lo_base, hi_base):
            e0 = unpack_acc([zero] * NACC,
                            row_chunks(rows_v, lo_base, hi_base, 0))

            def group(first):
                def gbody(r, accs):
                    return tuple(unpack_acc(accs,
                                            row_chunks(rows_v, lo_base,
                                                       hi_base, first + r)))
                return lax.fori_loop(0, S, gbody, tuple([zero] * NACC))

            g1 = group(1)
            g2 = group(1 + S)
            g3 = group(1 + 2 * S)
            return [w0 * a + w1 * b + w2 * c + w3 * d
                    for a, b, c, d in zip(e0, g1, g2, g3)]

        def compute(n, rows_v):
            u = side_repr(rows_v, 0, RW2)
            v = side_repr(rows_v, RW, RW2 + RW)
            p = u[0] * v[0]
            for c in range(1, NACC):
                p = p + u[c] * v[c]
            dot = jnp.sum(p)
            plsc.store_scatter(out_v, [jnp.full((L,), n, jnp.int32)],
                               jnp.full((L,), dot, jnp.float32),
                               mask=lanes == 0)

        issue(0, rows_a, sem_a)
        issue(1, rows_b, sem_b)

        def grp(g, carry):
            for n, rows_v, sem in ((2 * g, rows_a, sem_a),
                                   (2 * g + 1, rows_b, sem_b)):
                drain(rows_v, sem)
                compute(n, rows_v)

                @pl.when(n + 2 < PER_W)
                def _():
                    issue(n + 2, rows_v, sem)
            return carry

        lax.fori_loop(0, PER_W // 2, grp, 0)
        pltpu.sync_copy(out_v, out_hbm.at[wid])

    return body(tlo, thi, wpad, lists)


def _pack_lists(ind, walks):
    wt = walks.transpose(0, 2, 1)
    w75 = wt[:, 1:, :].reshape(B, S * K)
    pad = wt[:, 0, :4]
    return jnp.concatenate([ind[:, None], w75, pad], axis=1)


def kernel(node_embeddings, user_weights, item_weights,
           user_indices, item_indices, user_walks, item_walks):
    del item_weights
    wpad = jnp.pad(user_weights, (0, L - user_weights.shape[0]),
                   constant_values=-jnp.inf)
    zerof = user_weights[0] * jnp.float32(0.0)
    sel = np.zeros((D, 2 * DLO), np.float32)
    sel[np.arange(D), np.arange(D)] = 1.0
    sel_lo = jnp.asarray(sel[:, :DLO]) + zerof
    sel_hi = jnp.asarray(sel[:, DLO:]) + zerof
    dn = (((1,), (0,)), ((), ()))
    tlo = lax.dot_general(node_embeddings, sel_lo, dimension_numbers=dn,
                          preferred_element_type=jnp.bfloat16)
    thi = lax.dot_general(node_embeddings, sel_hi, dimension_numbers=dn,
                          preferred_element_type=jnp.bfloat16)
    ulists = _pack_lists(user_indices.astype(jnp.int32),
                         user_walks.astype(jnp.int32))
    ilists = _pack_lists(item_indices.astype(jnp.int32),
                         item_walks.astype(jnp.int32))
    lists = jnp.concatenate([ulists, ilists], axis=1).reshape(NW, PER_W, RW2)
    out = _sc_scores(tlo, thi, wpad, lists)
    return out.reshape(B)
